# Initial kernel scaffold; baseline (speedup 1.0000x reference)
import jax
import jax.numpy as jnp
from jax.experimental import pallas as pl


def kernel(xyz, points, W0, b0, g0, be0, W1, b1, g1, be1, W2, b2, g2, be2):
    raise NotImplementedError("write your pallas kernel here")


# speedup vs baseline: 2.7086x; 2.7086x over previous
import functools

import jax
import jax.numpy as jnp
from jax import lax
from jax.experimental import pallas as pl
from jax.experimental.pallas import tpu as pltpu
from jax.experimental.pallas import tpu_sc as plsc

_NPOINT = 512
_NSAMPLE = 32
_EPS = 1e-5


def _fps_body(x_ref, cents_ref, dd_ref):
    B = x_ref.shape[0]
    R, L = x_ref.shape[2], x_ref.shape[3]
    X = x_ref[:, 0]
    Y = x_ref[:, 1]
    Z = x_ref[:, 2]
    dd_ref[...] = (X * X + Y * Y) + Z * Z
    li = (lax.broadcasted_iota(jnp.int32, (B, R, L), 1) * L
          + lax.broadcasted_iota(jnp.int32, (B, R, L), 2))
    sel_i = lax.broadcasted_iota(jnp.int32, (1, 1, _NPOINT), 2)

    def body(i, carry):
        dist, f = carry
        Xv = x_ref[:, 0]
        Yv = x_ref[:, 1]
        Zv = x_ref[:, 2]
        mask = li == f
        cx = jnp.sum(jnp.sum(jnp.where(mask, Xv, 0.0), axis=2, keepdims=True),
                     axis=1, keepdims=True)
        cy = jnp.sum(jnp.sum(jnp.where(mask, Yv, 0.0), axis=2, keepdims=True),
                     axis=1, keepdims=True)
        cz = jnp.sum(jnp.sum(jnp.where(mask, Zv, 0.0), axis=2, keepdims=True),
                     axis=1, keepdims=True)
        ssc = (cx * cx + cy * cy) + cz * cz
        c4 = jnp.concatenate([cx, cy, cz, ssc], axis=1)
        cents_ref[...] = jnp.where(sel_i == i, c4, cents_ref[...])
        dx = Xv - cx
        dy = Yv - cy
        dz = Zv - cz
        d = (dx * dx + dy * dy) + dz * dz
        dist = jnp.minimum(dist, d)
        m = jnp.max(jnp.max(dist, axis=2, keepdims=True), axis=1, keepdims=True)
        f2 = jnp.min(jnp.min(jnp.where(dist == m, li, 2**30), axis=2,
                             keepdims=True), axis=1, keepdims=True)
        return dist, f2

    init = (jnp.full((B, R, L), 1e10, jnp.float32),
            jnp.zeros((B, 1, 1), jnp.int32))
    lax.fori_loop(0, _NPOINT, body, init)


def _run_fps(xyz):
    B, _, N = xyz.shape
    xr = xyz.reshape(B, 3, N // 128, 128)
    return pl.pallas_call(
        _fps_body,
        out_shape=[
            jax.ShapeDtypeStruct((B, 4, _NPOINT), jnp.float32),
            jax.ShapeDtypeStruct((B, N // 128, 128), jnp.float32),
        ],
    )(xr)


def _make_sc_gather(B, N, n_rows):
    nw = 32
    cpw = (B * _NPOINT) // nw
    mesh = plsc.VectorSubcoreMesh(core_axis_name="c", subcore_axis_name="s",
                                  num_cores=2, num_subcores=16)

    @functools.partial(
        pl.kernel,
        out_type=jax.ShapeDtypeStruct((n_rows, 32), jnp.float32),
        mesh=mesh,
        compiler_params=pltpu.CompilerParams(needs_layout_passes=False),
        scratch_types=[
            pltpu.VMEM((cpw, 16), jnp.float32),
            pltpu.VMEM((_NSAMPLE,), jnp.int32),
            pltpu.VMEM((_NSAMPLE, 128), jnp.float32),
            pltpu.VMEM((_NSAMPLE, 32), jnp.float32),
            pltpu.SemaphoreType.DMA,
        ],
    )
    def sc_gather(idx_hbm, cent_hbm, feat_hbm, out_hbm,
                  cent_v, idx_v, rows_v, rowso_v, sem):
        wid = lax.axis_index("s") * 2 + lax.axis_index("c")
        c0 = wid * cpw
        pltpu.sync_copy(cent_hbm.at[pl.ds(c0, cpw)], cent_v)
        lane = lax.broadcasted_iota(jnp.int32, (16,), 0)

        def per_center(j, carry):
            cidx = c0 + j
            pltpu.sync_copy(idx_hbm.at[cidx], idx_v)
            pltpu.async_copy(feat_hbm.at[idx_v], rows_v, sem).wait()
            crow = cent_v[j]
            sv = jnp.where(lane == 0, crow[0],
                           jnp.where(lane == 1, crow[1],
                                     jnp.where(lane == 2, crow[2], 0.0)))
            for r in range(_NSAMPLE):
                rowso_v[r, pl.ds(0, 16)] = rows_v[r, pl.ds(0, 16)] - sv
                rowso_v[r, pl.ds(16, 16)] = rows_v[r, pl.ds(16, 16)]
            pltpu.sync_copy(rowso_v, out_hbm.at[pl.ds(cidx * _NSAMPLE,
                                                      _NSAMPLE)])
            return carry

        lax.fori_loop(0, cpw, per_center, jnp.int32(0))

    return sc_gather


def _lin_body(x_ref, w_ref, p_ref, z_ref, st_ref, *, nc):
    x = x_ref[...]
    w = w_ref[...].astype(jnp.bfloat16)
    z = jnp.dot(x.astype(jnp.bfloat16), w,
                preferred_element_type=jnp.float32) + p_ref[0:1, :nc]
    z_ref[...] = z

    @pl.when(pl.program_id(0) == 0)
    def _():
        st_ref[...] = jnp.zeros_like(st_ref)

    st_ref[0:1, :nc] += jnp.sum(z, axis=0)[None, :]
    st_ref[1:2, :nc] += jnp.sum(z * z, axis=0)[None, :]


def _act_lin_body(x_ref, w_ref, p_ref, z_ref, st_ref, *, nc, nc_in):
    x = jnp.maximum(x_ref[...] * p_ref[1:2, :nc_in] + p_ref[2:3, :nc_in], 0.0)
    w = w_ref[...].astype(jnp.bfloat16)
    z = jnp.dot(x.astype(jnp.bfloat16), w,
                preferred_element_type=jnp.float32) + p_ref[0:1, :nc]
    z_ref[...] = z

    @pl.when(pl.program_id(0) == 0)
    def _():
        st_ref[...] = jnp.zeros_like(st_ref)

    st_ref[0:1, :nc] += jnp.sum(z, axis=0)[None, :]
    st_ref[1:2, :nc] += jnp.sum(z * z, axis=0)[None, :]


def _act_pool_body(x_ref, p_ref, o_ref, *, nc):
    y = jnp.maximum(x_ref[...] * p_ref[1:2, :nc] + p_ref[2:3, :nc], 0.0)
    g = y.shape[0] // _NSAMPLE
    o_ref[...] = jnp.max(y.reshape(g, _NSAMPLE, nc), axis=1)


def _run_lin(body, x, w, p, nc, blk, grid):
    m = x.shape[0]
    return pl.pallas_call(
        body,
        grid=(grid,),
        in_specs=[
            pl.BlockSpec((blk, x.shape[1]), lambda i: (i, 0)),
            pl.BlockSpec(w.shape, lambda i: (0, 0)),
            pl.BlockSpec(p.shape, lambda i: (0, 0)),
        ],
        out_specs=[
            pl.BlockSpec((blk, nc), lambda i: (i, 0)),
            pl.BlockSpec((8, 128), lambda i: (0, 0)),
        ],
        out_shape=[
            jax.ShapeDtypeStruct((m, nc), jnp.float32),
            jax.ShapeDtypeStruct((8, 128), jnp.float32),
        ],
    )(x, w, p)


def _bn_params(st, n, g, be, b_next, nc):
    s = st[0, :nc]
    q = st[1, :nc]
    mean = s / n
    var = q / n - mean * mean
    a = g / jnp.sqrt(var + _EPS)
    c = be - mean * a
    p = jnp.zeros((8, 128), jnp.float32)
    if b_next is not None:
        p = p.at[0, : b_next.shape[0]].set(b_next)
    p = p.at[1, :nc].set(a)
    p = p.at[2, :nc].set(c)
    return p


def kernel(xyz, points, W0, b0, g0, be0, W1, b1, g1, be1, W2, b2, g2, be2):
    B, _, N = xyz.shape
    D = points.shape[1]
    n_rows = B * _NPOINT * _NSAMPLE

    cents, dd = _run_fps(xyz)
    new_xyz = cents[:, :3, :]
    ss = cents[:, 3, :]
    dd = dd.reshape(B, N)

    xyz_t = xyz.transpose(0, 2, 1)
    new_t = new_xyz.transpose(0, 2, 1)
    sqrdists = -2.0 * jnp.matmul(new_t, xyz)
    sqrdists = sqrdists + jnp.sum(new_t ** 2, -1)[:, :, None]
    sqrdists = sqrdists + jnp.sum(xyz_t ** 2, -1)[:, None, :]
    gi = jnp.broadcast_to(jnp.arange(N, dtype=jnp.int32), sqrdists.shape)
    gi = jnp.where(sqrdists > 0.2 ** 2, jnp.int32(N), gi)
    gi = jnp.sort(gi, axis=-1)[:, :, :_NSAMPLE]
    gf = jnp.broadcast_to(gi[:, :, 0:1], gi.shape)
    gi = jnp.where(gi == N, gf, gi)
    gidx = (gi + (jnp.arange(B, dtype=jnp.int32) * N)[:, None, None]
            ).reshape(B * _NPOINT, _NSAMPLE)

    cent = jnp.concatenate(
        [new_xyz, jnp.zeros((B, 13, _NPOINT), jnp.float32)],
        axis=1).transpose(0, 2, 1).reshape(B * _NPOINT, 16)
    feat = jnp.concatenate(
        [xyz_t, points.transpose(0, 2, 1),
         jnp.zeros((B, N, 128 - 3 - D), jnp.float32)], axis=-1
    ).reshape(B * N, 128)

    sc = _make_sc_gather(B, N, n_rows)
    xg = sc(gidx, cent, feat)

    w0p = jnp.zeros((32, 32), jnp.float32).at[: 3 + D, :].set(W0.T)
    w1p = W1.T
    w2p = W2.T
    nf = float(n_rows)

    p1 = jnp.zeros((8, 128), jnp.float32).at[0, :32].set(b0)
    z1, st1 = _run_lin(functools.partial(_lin_body, nc=32), xg, w0p, p1,
                       32, 4096, n_rows // 4096)
    p2 = _bn_params(st1, nf, g0, be0, b1, 32)
    z2, st2 = _run_lin(functools.partial(_act_lin_body, nc=32, nc_in=32),
                       z1, w1p, p2, 32, 4096, n_rows // 4096)
    p3 = _bn_params(st2, nf, g1, be1, b2, 32)
    z3, st3 = _run_lin(functools.partial(_act_lin_body, nc=64, nc_in=32),
                       z2, w2p, p3, 64, 4096, n_rows // 4096)
    p4 = _bn_params(st3, nf, g2, be2, None, 64)

    blk = 4096
    pooled = pl.pallas_call(
        functools.partial(_act_pool_body, nc=64),
        grid=(n_rows // blk,),
        in_specs=[
            pl.BlockSpec((blk, 64), lambda i: (i, 0)),
            pl.BlockSpec((8, 128), lambda i: (0, 0)),
        ],
        out_specs=pl.BlockSpec((blk // _NSAMPLE, 64), lambda i: (i, 0)),
        out_shape=jax.ShapeDtypeStruct((B * _NPOINT, 64), jnp.float32),
    )(z3, p4)

    new_points = pooled.reshape(B, _NPOINT, 64).transpose(0, 2, 1)
    return (new_xyz, new_points)
